# Initial kernel scaffold; baseline (speedup 1.0000x reference)
import jax
import jax.numpy as jnp
from jax.experimental import pallas as pl


def kernel(x_atom, edge_index, edge_weight, batch, emb_0, emb_1, emb_2, emb_3, emb_4, emb_5, emb_6, emb_7, emb_8, W, Omega_1, V):
    raise NotImplementedError("write your pallas kernel here")


# speedup vs baseline: 4.1519x; 4.1519x over previous
import functools

import jax
import jax.numpy as jnp
import numpy as np
from jax import lax
from jax.experimental import pallas as pl
from jax.experimental.pallas import tpu as pltpu
from jax.experimental.pallas import tpu_sc as plsc

HID = 64
NN = 10000
NTILE = 32
RPT = 320
NP = NTILE * RPT
NE = 640000
CH = 128
EPAD = NE + 2 * CH
NG = 512
NCLASS = 2
KAPPA = 0.9
FW_ITERS = 30
ADIMS = [119, 5, 12, 12, 10, 6, 6, 2, 2]
AOFF = list(np.cumsum([0] + ADIMS[:-1]))
ETOT = 192
BLK = 1024
NBLK = NP // BLK


def _prep_body(xa_ref, emb_ref, om_ref, h_ref, y_ref):
    xa = xa_ref[...]
    et = emb_ref[...]
    cols = lax.broadcasted_iota(jnp.int32, (BLK, ETOT), 1)
    onehot = jnp.zeros((BLK, ETOT), jnp.float32)
    for i in range(9):
        idx = xa[:, i] % ADIMS[i] + AOFF[i]
        onehot = onehot + (idx[:, None] == cols).astype(jnp.float32)
    h = jnp.dot(onehot, et, preferred_element_type=jnp.float32, precision=lax.Precision.HIGHEST)
    h_ref[...] = h
    y_ref[...] = jnp.dot(h, om_ref[...].T, preferred_element_type=jnp.float32, precision=lax.Precision.HIGHEST)


def _proj_body(w_ref, wp_ref):
    w = w_ref[...]
    u = jnp.abs(w)
    su = jnp.sum(u, axis=1, keepdims=True)
    lo = jnp.zeros_like(su)
    hi = jnp.max(u, axis=1, keepdims=True)
    for _ in range(60):
        mid = 0.5 * (lo + hi)
        f = jnp.sum(jnp.maximum(u - mid, 0.0), axis=1, keepdims=True) - KAPPA
        pred = f > 0.0
        lo = jnp.where(pred, mid, lo)
        hi = jnp.where(pred, hi, mid)
    theta = 0.5 * (lo + hi)
    wrow = jnp.sign(w) * jnp.maximum(u - theta, 0.0)
    wp_ref[...] = jnp.where(su > KAPPA, wrow, w)


def _step_body(z_ref, b_ref, wp_ref, x_ref, y_ref):
    x = jnp.maximum(z_ref[...] + b_ref[...], 0.0)
    x_ref[...] = x
    y_ref[...] = jnp.dot(x, wp_ref[...].T, preferred_element_type=jnp.float32, precision=lax.Precision.HIGHEST)


def _step0_body(b_ref, wp_ref, y_ref):
    y_ref[...] = jnp.dot(b_ref[...], wp_ref[...].T,
                         preferred_element_type=jnp.float32, precision=lax.Precision.HIGHEST)


def _pool_body(x_ref, bt_ref, out_ref):
    @pl.when(pl.program_id(0) == 0)
    def _():
        out_ref[...] = jnp.zeros_like(out_ref)

    x = x_ref[...]
    nrm = jnp.sqrt(jnp.sum(x * x, axis=1, keepdims=True))
    xn = x / jnp.maximum(nrm, 1e-12)
    bt = bt_ref[0, 0, :]
    seg = lax.broadcasted_iota(jnp.int32, (NG, BLK), 0)
    oh = (seg == bt[None, :]).astype(jnp.float32)
    out_ref[...] += jnp.dot(oh, xn, preferred_element_type=jnp.float32, precision=lax.Precision.HIGHEST)


def _head_body(p_ref, v_ref, out_ref):
    out_ref[...] = lax.dot_general(
        p_ref[...], v_ref[...], (((1,), (1,)), ((), ())),
        preferred_element_type=jnp.float32, precision=lax.Precision.HIGHEST)


_prep_call = pl.pallas_call(
    _prep_body,
    grid=(NBLK,),
    in_specs=[
        pl.BlockSpec((BLK, 16), lambda i: (i, 0)),
        pl.BlockSpec((ETOT, HID), lambda i: (0, 0)),
        pl.BlockSpec((HID, HID), lambda i: (0, 0)),
    ],
    out_specs=[
        pl.BlockSpec((BLK, HID), lambda i: (i, 0)),
        pl.BlockSpec((BLK, HID), lambda i: (i, 0)),
    ],
    out_shape=[
        jax.ShapeDtypeStruct((NP, HID), jnp.float32),
        jax.ShapeDtypeStruct((NP, HID), jnp.float32),
    ],
)

_proj_call = pl.pallas_call(
    _proj_body,
    out_shape=jax.ShapeDtypeStruct((HID, HID), jnp.float32),
)

_step_call = pl.pallas_call(
    _step_body,
    grid=(NBLK,),
    in_specs=[
        pl.BlockSpec((BLK, HID), lambda i: (i, 0)),
        pl.BlockSpec((BLK, HID), lambda i: (i, 0)),
        pl.BlockSpec((HID, HID), lambda i: (0, 0)),
    ],
    out_specs=[
        pl.BlockSpec((BLK, HID), lambda i: (i, 0)),
        pl.BlockSpec((BLK, HID), lambda i: (i, 0)),
    ],
    out_shape=[
        jax.ShapeDtypeStruct((NP, HID), jnp.float32),
        jax.ShapeDtypeStruct((NP, HID), jnp.float32),
    ],
)

_step0_call = pl.pallas_call(
    _step0_body,
    grid=(NBLK,),
    in_specs=[
        pl.BlockSpec((BLK, HID), lambda i: (i, 0)),
        pl.BlockSpec((HID, HID), lambda i: (0, 0)),
    ],
    out_specs=pl.BlockSpec((BLK, HID), lambda i: (i, 0)),
    out_shape=jax.ShapeDtypeStruct((NP, HID), jnp.float32),
)

_pool_call = pl.pallas_call(
    _pool_body,
    grid=(NBLK,),
    in_specs=[
        pl.BlockSpec((BLK, HID), lambda i: (i, 0)),
        pl.BlockSpec((1, 1, BLK), lambda i: (i, 0, 0)),
    ],
    out_specs=pl.BlockSpec((NG, HID), lambda i: (0, 0)),
    out_shape=jax.ShapeDtypeStruct((NG, HID), jnp.float32),
)

_head_call = pl.pallas_call(
    _head_body,
    out_shape=jax.ShapeDtypeStruct((NG, NCLASS), jnp.float32),
)


_mesh = plsc.VectorSubcoreMesh(core_axis_name="c", subcore_axis_name="s")


@functools.partial(
    pl.kernel,
    mesh=_mesh,
    compiler_params=pltpu.CompilerParams(
        needs_layout_passes=False, use_tc_tiling_on_sc=False),
    out_type=jax.ShapeDtypeStruct((NP, HID), jnp.float32),
    scratch_types=[
        pltpu.VMEM((RPT, HID), jnp.float32),
        pltpu.VMEM((CH, HID), jnp.float32),
        pltpu.VMEM((CH,), jnp.int32),
        pltpu.VMEM((CH,), jnp.int32),
        pltpu.VMEM((CH,), jnp.float32),
        pltpu.VMEM((128,), jnp.int32),
        pltpu.SemaphoreType.DMA,
    ],
)
def _spmm_kernel(y_hbm, row_hbm, col_hbm, w_hbm, eoff_hbm, zero_hbm, z_hbm,
                 acc, rows, ridx, cidx, wbuf, eoffv, sem):
    c = lax.axis_index("c")
    s = lax.axis_index("s")
    t = s * 2 + c
    lane = jnp.arange(16, dtype=jnp.int32)

    pltpu.sync_copy(zero_hbm, acc)
    pltpu.sync_copy(eoff_hbm, eoffv)
    e0 = jnp.max(plsc.load_gather(eoffv, [jnp.full((16,), t, jnp.int32)]))
    e1 = jnp.max(plsc.load_gather(eoffv, [jnp.full((16,), t + 1, jnp.int32)]))
    a0 = (e0 // CH) * CH
    nch = (e1 - a0 + CH - 1) // CH

    def chunk(k, carry):
        sg = a0 + k * CH
        pltpu.sync_copy(row_hbm.at[pl.ds(sg, CH)], ridx)
        pltpu.sync_copy(col_hbm.at[pl.ds(sg, CH)], cidx)
        pltpu.sync_copy(w_hbm.at[pl.ds(sg, CH)], wbuf)
        pltpu.async_copy(y_hbm.at[ridx], rows, sem).wait()
        for v in range(CH // 16):
            gp = sg + v * 16 + lane
            wv = wbuf[pl.ds(v * 16, 16)]
            wbuf[pl.ds(v * 16, 16)] = jnp.where(
                (gp >= e0) & (gp < e1), wv, 0.0)

        def edge(e, carry2):
            ev = jnp.full((16,), e, jnp.int32)
            wv = plsc.load_gather(wbuf, [ev])
            cv = plsc.load_gather(cidx, [ev])
            for f in range(4):
                fl = lane + f * 16
                rv = plsc.load_gather(rows, [ev, fl])
                plsc.addupdate_scatter(acc, [cv, fl], rv * wv)
            return carry2

        return lax.fori_loop(0, CH, edge, carry)

    lax.fori_loop(0, nch, chunk, 0)
    pltpu.sync_copy(acc, z_hbm.at[pl.ds(t * RPT, RPT)])


def kernel(x_atom, edge_index, edge_weight, batch, emb_0, emb_1, emb_2,
           emb_3, emb_4, emb_5, emb_6, emb_7, emb_8, W, Omega_1, V):
    embs = [emb_0, emb_1, emb_2, emb_3, emb_4, emb_5, emb_6, emb_7, emb_8]
    ecat = jnp.concatenate(embs, axis=0).astype(jnp.float32)
    ecat = jnp.pad(ecat, ((0, ETOT - ecat.shape[0]), (0, 0)))

    xa = jnp.pad(x_atom.astype(jnp.int32), ((0, NP - NN), (0, 16 - 9)))

    row = edge_index[0].astype(jnp.int32)
    col = edge_index[1].astype(jnp.int32)
    perm = jnp.argsort(col)
    row_s = row[perm]
    col_s = col[perm]
    w_s = edge_weight[perm].astype(jnp.float32)
    eoff = jnp.searchsorted(
        col_s, jnp.arange(33, dtype=jnp.int32) * RPT).astype(jnp.int32)
    eoff = jnp.pad(eoff, (0, 128 - 33))
    row_p = jnp.pad(row_s, (0, EPAD - NE))
    col_l = jnp.pad(col_s % RPT, (0, EPAD - NE))
    w_p = jnp.pad(w_s, (0, EPAD - NE))

    batch_p = jnp.concatenate(
        [batch.astype(jnp.int32),
         jnp.full((NP - NN,), NG - 1, jnp.int32)]).reshape(NBLK, 1, BLK)

    zero_acc = jnp.zeros((RPT, HID), jnp.float32)

    h, y0 = _prep_call(xa, ecat, Omega_1.astype(jnp.float32))
    wp = _proj_call(W.astype(jnp.float32))

    def spmm(y):
        return _spmm_kernel(y, row_p, col_l, w_p, eoff, zero_acc)

    bres = spmm(y0)
    y1 = _step0_call(bres, wp)

    def body(i, carry):
        x, y = carry
        z = spmm(y)
        x2, y2 = _step_call(z, bres, wp)
        return (x2, y2)

    x, _ = lax.fori_loop(0, FW_ITERS, body, (bres, y1))

    pooled = _pool_call(x, batch_p)
    return _head_call(pooled, V.astype(jnp.float32))


# speedup vs baseline: 14.6576x; 3.5304x over previous
import functools

import jax
import jax.numpy as jnp
import numpy as np
from jax import lax
from jax.experimental import pallas as pl
from jax.experimental.pallas import tpu as pltpu
from jax.experimental.pallas import tpu_sc as plsc

HID = 64
NN = 10000
NTILE = 32
RPT = 320
NP = NTILE * RPT
NE = 640000
CH = 128
SB = 16
EPAD = NE + 2 * SB * CH
ECH = EPAD // CH
NG = 512
NCLASS = 2
KAPPA = 0.9
FW_ITERS = 30
ADIMS = [119, 5, 12, 12, 10, 6, 6, 2, 2]
AOFF = list(np.cumsum([0] + ADIMS[:-1]))
ETOT = 192
BLK = 1024
NBLK = NP // BLK


def _prep_body(xa_ref, emb_ref, om_ref, h_ref, y_ref):
    xa = xa_ref[...]
    et = emb_ref[...]
    cols = lax.broadcasted_iota(jnp.int32, (BLK, ETOT), 1)
    onehot = jnp.zeros((BLK, ETOT), jnp.float32)
    for i in range(9):
        idx = xa[:, i] % ADIMS[i] + AOFF[i]
        onehot = onehot + (idx[:, None] == cols).astype(jnp.float32)
    h = jnp.dot(onehot, et, preferred_element_type=jnp.float32, precision=lax.Precision.HIGHEST)
    h_ref[...] = h
    y_ref[...] = jnp.dot(h, om_ref[...].T, preferred_element_type=jnp.float32)


def _proj_body(w_ref, wp_ref):
    w = w_ref[...]
    u = jnp.abs(w)
    su = jnp.sum(u, axis=1, keepdims=True)
    lo = jnp.zeros_like(su)
    hi = jnp.max(u, axis=1, keepdims=True)
    for _ in range(60):
        mid = 0.5 * (lo + hi)
        f = jnp.sum(jnp.maximum(u - mid, 0.0), axis=1, keepdims=True) - KAPPA
        pred = f > 0.0
        lo = jnp.where(pred, mid, lo)
        hi = jnp.where(pred, hi, mid)
    theta = 0.5 * (lo + hi)
    wrow = jnp.sign(w) * jnp.maximum(u - theta, 0.0)
    wp_ref[...] = jnp.where(su > KAPPA, wrow, w)


def _step_body(z_ref, b_ref, wp_ref, x_ref, y_ref):
    x = jnp.maximum(z_ref[...] + b_ref[...], 0.0)
    x_ref[...] = x
    y_ref[...] = jnp.dot(x, wp_ref[...].T, preferred_element_type=jnp.float32)


def _step0_body(b_ref, wp_ref, y_ref):
    y_ref[...] = jnp.dot(b_ref[...], wp_ref[...].T,
                         preferred_element_type=jnp.float32)


def _pool_body(x_ref, bt_ref, out_ref):
    @pl.when(pl.program_id(0) == 0)
    def _():
        out_ref[...] = jnp.zeros_like(out_ref)

    x = x_ref[...]
    nrm = jnp.sqrt(jnp.sum(x * x, axis=1, keepdims=True))
    xn = x / jnp.maximum(nrm, 1e-12)
    bt = bt_ref[0, 0, :]
    seg = lax.broadcasted_iota(jnp.int32, (NG, BLK), 0)
    oh = (seg == bt[None, :]).astype(jnp.float32)
    out_ref[...] += jnp.dot(oh, xn, preferred_element_type=jnp.float32, precision=lax.Precision.HIGHEST)


_prep_call = pl.pallas_call(
    _prep_body,
    grid=(NBLK,),
    in_specs=[
        pl.BlockSpec((BLK, 16), lambda i: (i, 0)),
        pl.BlockSpec((ETOT, HID), lambda i: (0, 0)),
        pl.BlockSpec((HID, HID), lambda i: (0, 0)),
    ],
    out_specs=[
        pl.BlockSpec((BLK, HID), lambda i: (i, 0)),
        pl.BlockSpec((BLK, HID), lambda i: (i, 0)),
    ],
    out_shape=[
        jax.ShapeDtypeStruct((NP, HID), jnp.float32),
        jax.ShapeDtypeStruct((NP, HID), jnp.float32),
    ],
)

_proj_call = pl.pallas_call(
    _proj_body,
    out_shape=jax.ShapeDtypeStruct((HID, HID), jnp.float32),
)

_step_call = pl.pallas_call(
    _step_body,
    grid=(NBLK,),
    in_specs=[
        pl.BlockSpec((BLK, HID), lambda i: (i, 0)),
        pl.BlockSpec((BLK, HID), lambda i: (i, 0)),
        pl.BlockSpec((HID, HID), lambda i: (0, 0)),
    ],
    out_specs=[
        pl.BlockSpec((BLK, HID), lambda i: (i, 0)),
        pl.BlockSpec((BLK, HID), lambda i: (i, 0)),
    ],
    out_shape=[
        jax.ShapeDtypeStruct((NP, HID), jnp.float32),
        jax.ShapeDtypeStruct((NP, HID), jnp.float32),
    ],
)

_step0_call = pl.pallas_call(
    _step0_body,
    grid=(NBLK,),
    in_specs=[
        pl.BlockSpec((BLK, HID), lambda i: (i, 0)),
        pl.BlockSpec((HID, HID), lambda i: (0, 0)),
    ],
    out_specs=pl.BlockSpec((BLK, HID), lambda i: (i, 0)),
    out_shape=jax.ShapeDtypeStruct((NP, HID), jnp.float32),
)

_pool_call = pl.pallas_call(
    _pool_body,
    grid=(NBLK,),
    in_specs=[
        pl.BlockSpec((BLK, HID), lambda i: (i, 0)),
        pl.BlockSpec((1, 1, BLK), lambda i: (i, 0, 0)),
    ],
    out_specs=pl.BlockSpec((NG, HID), lambda i: (0, 0)),
    out_shape=jax.ShapeDtypeStruct((NG, HID), jnp.float32),
)


_mesh = plsc.VectorSubcoreMesh(core_axis_name="c", subcore_axis_name="s")


@functools.partial(
    pl.kernel,
    mesh=_mesh,
    compiler_params=pltpu.CompilerParams(
        needs_layout_passes=False, use_tc_tiling_on_sc=False),
    out_type=jax.ShapeDtypeStruct((NP, HID), jnp.float32),
    scratch_types=[
        pltpu.VMEM((RPT, HID), jnp.float32),
        pltpu.VMEM((CH, HID), jnp.float32),
        pltpu.VMEM((CH, HID), jnp.float32),
        pltpu.VMEM((SB, CH), jnp.int32),
        pltpu.VMEM((SB, CH), jnp.int32),
        pltpu.VMEM((SB, CH), jnp.float32),
        pltpu.VMEM((128,), jnp.int32),
        pltpu.SemaphoreType.DMA,
        pltpu.SemaphoreType.DMA,
    ],
)
def _spmm_kernel(y_hbm, row_hbm, col_hbm, w_hbm, eoff_hbm, zero_hbm, z_hbm,
                 acc, rows0, rows1, ridx, cidx, wbuf, eoffv, sem0, sem1):
    c = lax.axis_index("c")
    s = lax.axis_index("s")
    t = s * 2 + c
    lane = jnp.arange(16, dtype=jnp.int32)

    pltpu.sync_copy(zero_hbm, acc)
    pltpu.sync_copy(eoff_hbm, eoffv)
    e0 = jnp.max(plsc.load_gather(eoffv, [jnp.full((16,), t, jnp.int32)]))
    e1 = jnp.max(plsc.load_gather(eoffv, [jnp.full((16,), t + 1, jnp.int32)]))
    a0 = (e0 // CH) * CH
    nch = (e1 - a0 + CH - 1) // CH
    nsb = (nch + SB - 1) // SB
    mrow0 = a0 // CH
    rbufs = (rows0, rows1)
    sems = (sem0, sem1)

    def superblock(sb, carry):
        mr = mrow0 + sb * SB
        pltpu.sync_copy(row_hbm.at[pl.ds(mr, SB)], ridx)
        pltpu.sync_copy(col_hbm.at[pl.ds(mr, SB)], cidx)
        pltpu.sync_copy(w_hbm.at[pl.ds(mr, SB)], wbuf)
        base = a0 + sb * (SB * CH)
        for j in range(SB):
            for v in range(CH // 16):
                gp = base + j * CH + v * 16 + lane
                wv = wbuf[j, pl.ds(v * 16, 16)]
                wbuf[j, pl.ds(v * 16, 16)] = jnp.where(
                    (gp >= e0) & (gp < e1), wv, 0.0)

        def process(j, buf):
            @plsc.parallel_loop(0, CH, 1, unroll=4)
            def edge(e):
                ev = jnp.full((16,), e, jnp.int32)
                jv = jnp.full((16,), j, jnp.int32)
                wv = plsc.load_gather(wbuf, [jv, ev])
                cv = plsc.load_gather(cidx, [jv, ev])
                for f in range(4):
                    fl = lane + f * 16
                    rv = buf[e, pl.ds(f * 16, 16)]
                    plsc.addupdate_scatter(acc, [cv, fl], rv * wv)

        pending = [None, None]
        pending[0] = pltpu.async_copy(y_hbm.at[ridx.at[0]], rbufs[0], sems[0])
        for j in range(SB):
            nj = j + 1
            if nj < SB:
                pending[nj % 2] = pltpu.async_copy(
                    y_hbm.at[ridx.at[nj]], rbufs[nj % 2], sems[nj % 2])
            pending[j % 2].wait()
            process(j, rbufs[j % 2])
        return carry

    lax.fori_loop(0, nsb, superblock, 0)
    pltpu.sync_copy(acc, z_hbm.at[pl.ds(t * RPT, RPT)])


def kernel(x_atom, edge_index, edge_weight, batch, emb_0, emb_1, emb_2,
           emb_3, emb_4, emb_5, emb_6, emb_7, emb_8, W, Omega_1, V):
    embs = [emb_0, emb_1, emb_2, emb_3, emb_4, emb_5, emb_6, emb_7, emb_8]
    ecat = jnp.concatenate(embs, axis=0).astype(jnp.float32)
    ecat = jnp.pad(ecat, ((0, ETOT - ecat.shape[0]), (0, 0)))

    xa = jnp.pad(x_atom.astype(jnp.int32), ((0, NP - NN), (0, 16 - 9)))

    row = edge_index[0].astype(jnp.int32)
    col = edge_index[1].astype(jnp.int32)
    perm = jnp.argsort(col)
    row_s = row[perm]
    col_s = col[perm]
    w_s = edge_weight[perm].astype(jnp.float32)
    eoff = jnp.searchsorted(
        col_s, jnp.arange(33, dtype=jnp.int32) * RPT).astype(jnp.int32)
    eoff = jnp.pad(eoff, (0, 128 - 33))
    row_p = jnp.pad(row_s, (0, EPAD - NE)).reshape(ECH, CH)
    col_l = jnp.pad(col_s % RPT, (0, EPAD - NE)).reshape(ECH, CH)
    w_p = jnp.pad(w_s, (0, EPAD - NE)).reshape(ECH, CH)

    batch_p = jnp.concatenate(
        [batch.astype(jnp.int32),
         jnp.full((NP - NN,), NG - 1, jnp.int32)]).reshape(NBLK, 1, BLK)

    zero_acc = jnp.zeros((RPT, HID), jnp.float32)

    h, y0 = _prep_call(xa, ecat, Omega_1.astype(jnp.float32))
    wp = _proj_call(W.astype(jnp.float32))

    def spmm(y):
        return _spmm_kernel(y, row_p, col_l, w_p, eoff, zero_acc)

    bres = spmm(y0)
    y1 = _step0_call(bres, wp)

    def body(i, carry):
        x, y = carry
        z = spmm(y)
        x2, y2 = _step_call(z, bres, wp)
        return (x2, y2)

    x, _ = lax.fori_loop(0, FW_ITERS, body, (bres, y1))

    pooled = _pool_call(x, batch_p)
    return pooled @ V.astype(jnp.float32).T
